# baseline (speedup 1.0000x reference)
import functools

import jax
import jax.numpy as jnp
from jax import lax
from jax.experimental import pallas as pl
from jax.experimental.pallas import tpu as pltpu
from jax.experimental.pallas import tpu_sc as plsc

_INFO = plsc.get_sparse_core_info()
_NC = _INFO.num_cores
_NS = _INFO.num_subcores
_L = _INFO.num_lanes
_NW = _NC * _NS

_R = 1024
_C = 32768
_RPW = _R // _NW


@functools.partial(
    pl.kernel,
    mesh=plsc.VectorSubcoreMesh(core_axis_name="c", subcore_axis_name="s"),
    out_type=jax.ShapeDtypeStruct((_R, _C), jnp.float32),
    compiler_params=pltpu.CompilerParams(needs_layout_passes=False),
    scratch_types=[
        pltpu.VMEM((_C,), jnp.float32),
        pltpu.VMEM((_L,), jnp.float32),
        pltpu.VMEM((_L,), jnp.int32),
    ],
)
def _hardmax_rows(x_hbm, out_hbm, buf, lane_v, lane_i):
    wid = lax.axis_index("s") * _NC + lax.axis_index("c")
    iota = lax.iota(jnp.int32, _L)
    zeros = jnp.zeros((_L,), jnp.float32)

    def do_row(i, carry):
        row = wid * _RPW + i
        pltpu.sync_copy(x_hbm.at[row], buf)

        def chunk(k, st):
            best, bidx = st
            off = k * _L
            v = buf[pl.ds(off, _L)]
            m = v > best
            best = jnp.where(m, v, best)
            bidx = jnp.where(m, off + iota, bidx)
            buf[pl.ds(off, _L)] = zeros
            return best, bidx

        best, bidx = lax.fori_loop(
            0, _C // _L, chunk,
            (jnp.full((_L,), -jnp.inf, jnp.float32),
             jnp.zeros((_L,), jnp.int32)))

        cur_v, cur_i = best, bidx
        for shift in (8, 4, 2, 1):
            lane_v[...] = cur_v
            lane_i[...] = cur_i
            g = jnp.minimum(iota + shift, jnp.int32(_L - 1))
            o_v = plsc.load_gather(lane_v, [g])
            o_i = plsc.load_gather(lane_i, [g])
            better = (o_v > cur_v) | ((o_v == cur_v) & (o_i < cur_i))
            cur_v = jnp.where(better, o_v, cur_v)
            cur_i = jnp.where(better, o_i, cur_i)
        plsc.store_scatter(buf, [cur_i], cur_v, mask=iota == 0)
        pltpu.sync_copy(buf, out_hbm.at[row])
        return carry

    lax.fori_loop(0, _RPW, do_row, 0)


def kernel(x):
    out = _hardmax_rows(x.reshape(_R, _C))
    return out.reshape(64, 16, _C)


# speedup vs baseline: 1.3193x; 1.3193x over previous
import functools

import jax
import jax.numpy as jnp
from jax import lax
from jax.experimental import pallas as pl
from jax.experimental.pallas import tpu as pltpu
from jax.experimental.pallas import tpu_sc as plsc

_INFO = plsc.get_sparse_core_info()
_NC = _INFO.num_cores
_NS = _INFO.num_subcores
_L = _INFO.num_lanes
_NW = _NC * _NS

_R = 1024
_C = 32768
_RPW = _R // _NW


@functools.partial(
    pl.kernel,
    mesh=plsc.VectorSubcoreMesh(core_axis_name="c", subcore_axis_name="s"),
    out_type=jax.ShapeDtypeStruct((_R, _C), jnp.float32),
    compiler_params=pltpu.CompilerParams(needs_layout_passes=False),
    scratch_types=[
        pltpu.VMEM((_C,), jnp.float32),
        pltpu.VMEM((_C,), jnp.float32),
        pltpu.VMEM((_L,), jnp.float32),
        pltpu.VMEM((_L,), jnp.int32),
        pltpu.SemaphoreType.DMA,
        pltpu.SemaphoreType.DMA,
        pltpu.SemaphoreType.DMA,
        pltpu.SemaphoreType.DMA,
    ],
)
def _hardmax_rows(x_hbm, out_hbm, buf0, buf1, lane_v, lane_i,
                  gsem0, gsem1, osem0, osem1):
    wid = lax.axis_index("s") * _NC + lax.axis_index("c")
    base = wid * _RPW
    bufs = (buf0, buf1)
    gsems = (gsem0, gsem1)
    osems = (osem0, osem1)
    iota = lax.iota(jnp.int32, _L)
    zeros = jnp.zeros((_L,), jnp.float32)

    def compute_row(buf):
        def chunk(k, st):
            best, bidx, idxv = st
            off = k * _L
            v = buf[pl.ds(off, _L)]
            m = v > best
            best = jnp.where(m, v, best)
            bidx = jnp.where(m, idxv, bidx)
            buf[pl.ds(off, _L)] = zeros
            return best, bidx, idxv + _L

        best, bidx, _ = lax.fori_loop(
            0, _C // _L, chunk,
            (jnp.full((_L,), -jnp.inf, jnp.float32),
             jnp.zeros((_L,), jnp.int32),
             iota))

        cur_v, cur_i = best, bidx
        for shift in (8, 4, 2, 1):
            lane_v[...] = cur_v
            lane_i[...] = cur_i
            g = jnp.minimum(iota + shift, jnp.int32(_L - 1))
            o_v = plsc.load_gather(lane_v, [g])
            o_i = plsc.load_gather(lane_i, [g])
            better = (o_v > cur_v) | ((o_v == cur_v) & (o_i < cur_i))
            cur_v = jnp.where(better, o_v, cur_v)
            cur_i = jnp.where(better, o_i, cur_i)
        plsc.store_scatter(buf, [cur_i], cur_v, mask=iota == 0)

    gh = [None, None]
    oh = [None, None]
    g = pltpu.make_async_copy(x_hbm.at[base], bufs[0], gsems[0])
    g.start()
    gh[0] = g
    for i in range(_RPW):
        b = i % 2
        nb = 1 - b
        if i + 1 < _RPW:
            if i >= 1:
                oh[nb].wait()
            g = pltpu.make_async_copy(
                x_hbm.at[base + i + 1], bufs[nb], gsems[nb])
            g.start()
            gh[nb] = g
        gh[b].wait()
        compute_row(bufs[b])
        o = pltpu.make_async_copy(bufs[b], out_hbm.at[base + i], osems[b])
        o.start()
        oh[b] = o
    oh[0].wait()
    oh[1].wait()


def kernel(x):
    out = _hardmax_rows(x.reshape(_R, _C))
    return out.reshape(64, 16, _C)
